# baseline (speedup 1.0000x reference)
import functools

import jax
import jax.numpy as jnp
from jax import lax
from jax.experimental import pallas as pl
from jax.experimental.pallas import tpu as pltpu
from jax.experimental.pallas import tpu_sc as plsc

VOCAB = 100000
EMB = 64
BATCH = 1024

_INFO = plsc.get_sparse_core_info()
_NC = _INFO.num_cores
_NS = _INFO.num_subcores
_NW = _NC * _NS
_B_PER_W = BATCH // _NW


def _sc_gather(table, idx):
  mesh = plsc.VectorSubcoreMesh(core_axis_name="c", subcore_axis_name="s")

  @functools.partial(
      pl.kernel,
      mesh=mesh,
      out_type=jax.ShapeDtypeStruct((BATCH, EMB), jnp.float32),
      scratch_types=[
          pltpu.VMEM((_B_PER_W,), jnp.int32),
          pltpu.VMEM((_B_PER_W, EMB), jnp.float32),
          pltpu.SemaphoreType.DMA,
      ],
      compiler_params=pltpu.CompilerParams(use_tc_tiling_on_sc=False),
  )
  def gather_kernel(table_hbm, idx_hbm, out_hbm, idx_v, rows_v, sem):
    wid = lax.axis_index("s") * _NC + lax.axis_index("c")
    base = wid * _B_PER_W
    pltpu.sync_copy(idx_hbm.at[pl.ds(base, _B_PER_W)], idx_v)
    pltpu.async_copy(table_hbm.at[idx_v], rows_v, sem).wait()
    pltpu.sync_copy(rows_v, out_hbm.at[pl.ds(base, _B_PER_W)])

  return gather_kernel(table, idx)


_VBLK = 1024


def _mm_body(emb_ref, w_ref, b_ref, out_ref):
  out_ref[...] = lax.dot_general(
      emb_ref[...], w_ref[...],
      (((1,), (1,)), ((), ())),
      preferred_element_type=jnp.float32,
  ) + b_ref[...]


def _tc_project(emb, lin_w, lin_b2d):
  grid = (pl.cdiv(VOCAB, _VBLK),)
  return pl.pallas_call(
      _mm_body,
      grid=grid,
      in_specs=[
          pl.BlockSpec((BATCH, EMB), lambda j: (0, 0)),
          pl.BlockSpec((_VBLK, EMB), lambda j: (j, 0)),
          pl.BlockSpec((1, _VBLK), lambda j: (0, j)),
      ],
      out_specs=pl.BlockSpec((BATCH, _VBLK), lambda j: (0, j)),
      out_shape=jax.ShapeDtypeStruct((BATCH, VOCAB), jnp.float32),
  )(emb, lin_w, lin_b2d)


@jax.jit
def kernel(center_id, emb_table, lin_w, lin_b):
  emb = _sc_gather(emb_table, center_id.astype(jnp.int32))
  return _tc_project(emb, lin_w, lin_b.reshape(1, VOCAB))


# speedup vs baseline: 2.9180x; 2.9180x over previous
import functools

import jax
import jax.numpy as jnp
from jax import lax
from jax.experimental import pallas as pl
from jax.experimental.pallas import tpu as pltpu
from jax.experimental.pallas import tpu_sc as plsc

VOCAB = 100000
EMB = 64
BATCH = 1024

_INFO = plsc.get_sparse_core_info()
_NC = _INFO.num_cores
_NS = _INFO.num_subcores
_NW = _NC * _NS
_B_PER_W = BATCH // _NW


def _sc_gather(table, idx):
  mesh = plsc.VectorSubcoreMesh(core_axis_name="c", subcore_axis_name="s")

  @functools.partial(
      pl.kernel,
      mesh=mesh,
      out_type=jax.ShapeDtypeStruct((BATCH, EMB), jnp.float32),
      scratch_types=[
          pltpu.VMEM((_B_PER_W,), jnp.int32),
          pltpu.VMEM((_B_PER_W, EMB), jnp.float32),
          pltpu.SemaphoreType.DMA,
      ],
      compiler_params=pltpu.CompilerParams(use_tc_tiling_on_sc=False),
  )
  def gather_kernel(table_hbm, idx_hbm, out_hbm, idx_v, rows_v, sem):
    wid = lax.axis_index("s") * _NC + lax.axis_index("c")
    base = wid * _B_PER_W
    pltpu.sync_copy(idx_hbm.at[pl.ds(base, _B_PER_W)], idx_v)
    pltpu.async_copy(table_hbm.at[idx_v], rows_v, sem).wait()
    pltpu.sync_copy(rows_v, out_hbm.at[pl.ds(base, _B_PER_W)])

  return gather_kernel(table, idx)


_VBLK = 2048


def _mm_body(wt_ref, b_ref, embt_ref, out_ref):
  w_aug = jnp.concatenate([wt_ref[...], b_ref[...]], axis=0)
  out_ref[...] = lax.dot_general(
      w_aug, embt_ref[...],
      (((0,), (0,)), ((), ())),
      preferred_element_type=jnp.float32,
  )


def _tc_project(embt_aug, lin_w_t, lin_b2d):
  grid = (pl.cdiv(VOCAB, _VBLK),)
  return pl.pallas_call(
      _mm_body,
      grid=grid,
      in_specs=[
          pl.BlockSpec((EMB, _VBLK), lambda j: (0, j)),
          pl.BlockSpec((1, _VBLK), lambda j: (0, j)),
          pl.BlockSpec((EMB + 1, BATCH), lambda j: (0, 0)),
      ],
      out_specs=pl.BlockSpec((_VBLK, BATCH), lambda j: (j, 0)),
      out_shape=jax.ShapeDtypeStruct((VOCAB, BATCH), jnp.float32),
  )(lin_w_t, lin_b2d, embt_aug)


@jax.jit
def kernel(center_id, emb_table, lin_w, lin_b):
  emb = _sc_gather(emb_table, center_id.astype(jnp.int32))
  embt_aug = jnp.concatenate(
      [emb.T, jnp.ones((1, BATCH), jnp.float32)], axis=0)
  out_t = _tc_project(embt_aug, lin_w.T, lin_b.reshape(1, VOCAB))
  return out_t.T


# speedup vs baseline: 2.9594x; 1.0142x over previous
import functools

import jax
import jax.numpy as jnp
from jax import lax
from jax.experimental import pallas as pl
from jax.experimental.pallas import tpu as pltpu
from jax.experimental.pallas import tpu_sc as plsc

VOCAB = 100000
EMB = 64
BATCH = 1024

_INFO = plsc.get_sparse_core_info()
_NC = _INFO.num_cores
_NS = _INFO.num_subcores
_NW = _NC * _NS
_B_PER_W = BATCH // _NW


def _sc_gather(table, idx):
  mesh = plsc.VectorSubcoreMesh(core_axis_name="c", subcore_axis_name="s")

  @functools.partial(
      pl.kernel,
      mesh=mesh,
      out_type=jax.ShapeDtypeStruct((BATCH, EMB), jnp.float32),
      scratch_types=[
          pltpu.VMEM((_B_PER_W,), jnp.int32),
          pltpu.VMEM((_B_PER_W, EMB), jnp.float32),
          pltpu.SemaphoreType.DMA,
      ],
      compiler_params=pltpu.CompilerParams(use_tc_tiling_on_sc=False),
  )
  def gather_kernel(table_hbm, idx_hbm, out_hbm, idx_v, rows_v, sem):
    wid = lax.axis_index("s") * _NC + lax.axis_index("c")
    base = wid * _B_PER_W
    pltpu.sync_copy(idx_hbm.at[pl.ds(base, _B_PER_W)], idx_v)
    pltpu.async_copy(table_hbm.at[idx_v], rows_v, sem).wait()
    pltpu.sync_copy(rows_v, out_hbm.at[pl.ds(base, _B_PER_W)])

  return gather_kernel(table, idx)


_VBLK = 4096


def _mm_body(wt_ref, b_ref, embt_ref, out_ref):
  w_aug = jnp.concatenate([wt_ref[...], b_ref[...]], axis=0)
  out_ref[...] = lax.dot_general(
      w_aug, embt_ref[...],
      (((0,), (0,)), ((), ())),
      preferred_element_type=jnp.float32,
  )


def _tc_project(embt_aug, lin_w_t, lin_b2d):
  grid = (pl.cdiv(VOCAB, _VBLK),)
  return pl.pallas_call(
      _mm_body,
      grid=grid,
      in_specs=[
          pl.BlockSpec((EMB, _VBLK), lambda j: (0, j)),
          pl.BlockSpec((1, _VBLK), lambda j: (0, j)),
          pl.BlockSpec((EMB + 1, BATCH), lambda j: (0, 0)),
      ],
      out_specs=pl.BlockSpec((_VBLK, BATCH), lambda j: (j, 0)),
      out_shape=jax.ShapeDtypeStruct((VOCAB, BATCH), jnp.float32),
  )(lin_w_t, lin_b2d, embt_aug)


@jax.jit
def kernel(center_id, emb_table, lin_w, lin_b):
  emb = _sc_gather(emb_table, center_id.astype(jnp.int32))
  embt_aug = jnp.concatenate(
      [emb.T, jnp.ones((1, BATCH), jnp.float32)], axis=0)
  out_t = _tc_project(embt_aug, lin_w.T, lin_b.reshape(1, VOCAB))
  return out_t.T


# speedup vs baseline: 3.7424x; 1.2646x over previous
import functools

import jax
import jax.numpy as jnp
from jax import lax
from jax.experimental import pallas as pl
from jax.experimental.pallas import tpu as pltpu
from jax.experimental.pallas import tpu_sc as plsc

VOCAB = 100000
EMB = 64
BATCH = 1024

_INFO = plsc.get_sparse_core_info()
_NC = _INFO.num_cores
_NS = _INFO.num_subcores
_NW = _NC * _NS
_B_PER_W = BATCH // _NW


def _sc_gather_t(table_t, idx):
  mesh = plsc.VectorSubcoreMesh(core_axis_name="c", subcore_axis_name="s")

  nslot = 4

  @functools.partial(
      pl.kernel,
      mesh=mesh,
      out_type=jax.ShapeDtypeStruct((BATCH, EMB), jnp.float32),
      scratch_types=(
          [pltpu.VMEM((_B_PER_W,), jnp.int32)]
          + [pltpu.VMEM((EMB, 128), jnp.float32)] * nslot
          + [pltpu.VMEM((_B_PER_W, EMB), jnp.float32)]
          + [pltpu.SemaphoreType.DMA] * nslot
      ),
      compiler_params=pltpu.CompilerParams(needs_layout_passes=False),
  )
  def gather_kernel(table_hbm, idx_hbm, out_hbm, idx_v, *rest):
    bufs = rest[:nslot]
    out_loc = rest[nslot]
    sems = rest[nslot + 1:]
    wid = lax.axis_index("s") * _NC + lax.axis_index("c")
    base = wid * _B_PER_W
    pltpu.sync_copy(idx_hbm.at[pl.ds(base, _B_PER_W)], idx_v)

    lane16 = lax.iota(jnp.int32, 16)
    idx_scalars = []
    for chunk in range(_B_PER_W // 16):
      vec = idx_v[pl.ds(chunk * 16, 16)]
      for k in range(16):
        idx_scalars.append(jnp.max(jnp.where(lane16 == k, vec, 0)))

    def slab_copy(i, slot):
      v = idx_scalars[i]
      off = pl.multiple_of((v // 128) * 128, 128)
      return pltpu.make_async_copy(
          table_hbm.at[:, pl.ds(off, 128)], bufs[slot], sems[slot])

    for s in range(nslot):
      slab_copy(s, s).start()
    for i in range(_B_PER_W):
      slot = i % nslot
      slab_copy(i, slot).wait()
      v = idx_scalars[i]
      col = v - (v // 128) * 128
      col_v = jnp.full((16,), col, jnp.int32)
      for p in range(EMB // 16):
        rows = lane16 + p * 16
        vals = plsc.load_gather(bufs[slot], [rows, col_v])
        out_loc[i, pl.ds(16 * p, 16)] = vals
      if i + nslot < _B_PER_W:
        slab_copy(i + nslot, slot).start()

    pltpu.sync_copy(out_loc, out_hbm.at[pl.ds(base, _B_PER_W), :])

  return gather_kernel(table_t, idx)


_VBLK = 4096


def _mm_body(wt_ref, b_ref, embt_ref, out_ref):
  w_aug = jnp.concatenate([wt_ref[...], b_ref[...]], axis=0)
  out_ref[...] = lax.dot_general(
      w_aug, embt_ref[...],
      (((0,), (0,)), ((), ())),
      preferred_element_type=jnp.float32,
  )


def _tc_project(embt_aug, lin_w_t, lin_b2d):
  grid = (pl.cdiv(VOCAB, _VBLK),)
  return pl.pallas_call(
      _mm_body,
      grid=grid,
      in_specs=[
          pl.BlockSpec((EMB, _VBLK), lambda j: (0, j)),
          pl.BlockSpec((1, _VBLK), lambda j: (0, j)),
          pl.BlockSpec((EMB + 1, BATCH), lambda j: (0, 0)),
      ],
      out_specs=pl.BlockSpec((_VBLK, BATCH), lambda j: (j, 0)),
      out_shape=jax.ShapeDtypeStruct((VOCAB, BATCH), jnp.float32),
  )(lin_w_t, lin_b2d, embt_aug)


@jax.jit
def kernel(center_id, emb_table, lin_w, lin_b):
  emb = _sc_gather_t(emb_table.T, center_id.astype(jnp.int32))
  embt_aug = jnp.concatenate(
      [emb.T, jnp.ones((1, BATCH), jnp.float32)], axis=0)
  out_t = _tc_project(embt_aug, lin_w.T, lin_b.reshape(1, VOCAB))
  return out_t.T


# speedup vs baseline: 3.8108x; 1.0183x over previous
import functools

import jax
import jax.numpy as jnp
from jax import lax
from jax.experimental import pallas as pl
from jax.experimental.pallas import tpu as pltpu
from jax.experimental.pallas import tpu_sc as plsc

VOCAB = 100000
EMB = 64
BATCH = 1024

_INFO = plsc.get_sparse_core_info()
_NC = _INFO.num_cores
_NS = _INFO.num_subcores
_NW = _NC * _NS
_B_PER_W = BATCH // _NW


def _sc_gather_t(table_t, idx):
  mesh = plsc.VectorSubcoreMesh(core_axis_name="c", subcore_axis_name="s")

  nslot = 8

  @functools.partial(
      pl.kernel,
      mesh=mesh,
      out_type=jax.ShapeDtypeStruct((BATCH, EMB), jnp.float32),
      scratch_types=(
          [pltpu.VMEM((_B_PER_W,), jnp.int32)]
          + [pltpu.VMEM((EMB, 128), jnp.float32)] * nslot
          + [pltpu.VMEM((_B_PER_W, EMB), jnp.float32)]
          + [pltpu.SemaphoreType.DMA] * nslot
      ),
      compiler_params=pltpu.CompilerParams(needs_layout_passes=False),
  )
  def gather_kernel(table_hbm, idx_hbm, out_hbm, idx_v, *rest):
    bufs = rest[:nslot]
    out_loc = rest[nslot]
    sems = rest[nslot + 1:]
    wid = lax.axis_index("s") * _NC + lax.axis_index("c")
    base = wid * _B_PER_W
    pltpu.sync_copy(idx_hbm.at[pl.ds(base, _B_PER_W)], idx_v)

    lane16 = lax.iota(jnp.int32, 16)
    idx_scalars = []
    for chunk in range(_B_PER_W // 16):
      vec = idx_v[pl.ds(chunk * 16, 16)]
      for k in range(16):
        idx_scalars.append(jnp.max(jnp.where(lane16 == k, vec, 0)))

    def slab_copy(i, slot):
      v = idx_scalars[i]
      off = pl.multiple_of((v // 128) * 128, 128)
      return pltpu.make_async_copy(
          table_hbm.at[:, pl.ds(off, 128)], bufs[slot], sems[slot])

    for s in range(nslot):
      slab_copy(s, s).start()
    for i in range(_B_PER_W):
      slot = i % nslot
      slab_copy(i, slot).wait()
      v = idx_scalars[i]
      col = v - (v // 128) * 128
      col_v = jnp.full((16,), col, jnp.int32)
      for p in range(EMB // 16):
        rows = lane16 + p * 16
        vals = plsc.load_gather(bufs[slot], [rows, col_v])
        out_loc[i, pl.ds(16 * p, 16)] = vals
      if i + nslot < _B_PER_W:
        slab_copy(i + nslot, slot).start()

    pltpu.sync_copy(out_loc, out_hbm.at[pl.ds(base, _B_PER_W), :])

  return gather_kernel(table_t, idx)


_VBLK = 4096


def _mm_body(wt_ref, b_ref, embt_ref, out_ref):
  w_aug = jnp.concatenate([wt_ref[...], b_ref[...]], axis=0)
  out_ref[...] = lax.dot_general(
      w_aug, embt_ref[...],
      (((0,), (0,)), ((), ())),
      preferred_element_type=jnp.float32,
  )


def _tc_project(embt_aug, lin_w_t, lin_b2d):
  grid = (pl.cdiv(VOCAB, _VBLK),)
  return pl.pallas_call(
      _mm_body,
      grid=grid,
      in_specs=[
          pl.BlockSpec((EMB, _VBLK), lambda j: (0, j)),
          pl.BlockSpec((1, _VBLK), lambda j: (0, j)),
          pl.BlockSpec((EMB + 1, BATCH), lambda j: (0, 0)),
      ],
      out_specs=pl.BlockSpec((_VBLK, BATCH), lambda j: (j, 0)),
      out_shape=jax.ShapeDtypeStruct((VOCAB, BATCH), jnp.float32),
  )(lin_w_t, lin_b2d, embt_aug)


@jax.jit
def kernel(center_id, emb_table, lin_w, lin_b):
  emb = _sc_gather_t(emb_table.T, center_id.astype(jnp.int32))
  embt_aug = jnp.concatenate(
      [emb.T, jnp.ones((1, BATCH), jnp.float32)], axis=0)
  out_t = _tc_project(embt_aug, lin_w.T, lin_b.reshape(1, VOCAB))
  return out_t.T
